# baseline (speedup 1.0000x reference)
import functools

import jax
import jax.numpy as jnp
from jax import lax
from jax.experimental import pallas as pl
from jax.experimental.pallas import tpu as pltpu
from jax.experimental.pallas import tpu_sc as plsc

N = 10000
NC = 2
NS = 16
NW = NC * NS
CH = 128
K = 80
K2 = 2 * K
EP = NW * K * CH
RB = 632
NPAD = NS * RB
RBLK = 2000


def _edge_kernel_l1():
  mesh = plsc.VectorSubcoreMesh(
      core_axis_name="c", subcore_axis_name="s", num_cores=NC, num_subcores=NS)
  out_type = [jax.ShapeDtypeStruct((NC, NPAD, 32), jnp.float32),
              jax.ShapeDtypeStruct((NC, NPAD, 8), jnp.float32)]
  scratch = [
      pltpu.VMEM((K2, CH), jnp.int32),
      pltpu.VMEM((K2, CH), jnp.int32),
      pltpu.VMEM((CH, 32), jnp.float32),
      pltpu.VMEM((CH, 32), jnp.float32),
      pltpu.VMEM((CH, 32), jnp.float32),
      pltpu.VMEM((CH, 32), jnp.float32),
      pltpu.VMEM((CH, 8), jnp.float32),
      pltpu.VMEM_SHARED((NPAD, 32), jnp.float32),
      pltpu.VMEM_SHARED((NPAD, 32), jnp.float32),
      pltpu.VMEM_SHARED((NPAD, 8), jnp.float32),
      pltpu.SemaphoreType.DMA,
      pltpu.SemaphoreType.DMA,
      pltpu.SemaphoreType.DMA,
      pltpu.SemaphoreType.DMA,
  ]

  def body(tab_a, tab_b, srcp, dstp, z32, z1, ones_h,
           out_agg, out_cnt,
           src_v, dst_v, rows0, rows1, rows2, rows3, ones_v,
           agg_sh, tab_sh, cnt_sh, sem0, sem1, sem2, sem3):
    c = lax.axis_index("c")
    s = lax.axis_index("s")
    r0 = s * RB
    pltpu.sync_copy(z32.at[pl.ds(r0, RB), :], agg_sh.at[pl.ds(r0, RB), :])
    pltpu.sync_copy(z1.at[pl.ds(r0, RB), :], cnt_sh.at[pl.ds(r0, RB), :])
    pltpu.sync_copy(ones_h, ones_v)

    @pl.when(c == 0)
    def _():
      pltpu.sync_copy(tab_a.at[pl.ds(r0, RB), :], tab_sh.at[pl.ds(r0, RB), :])

    @pl.when(c == 1)
    def _():
      pltpu.sync_copy(tab_b.at[pl.ds(r0, RB), :], tab_sh.at[pl.ds(r0, RB), :])

    plsc.subcore_barrier()

    base = s * K2
    pltpu.sync_copy(srcp.at[pl.ds(base, K2), :], src_v)
    pltpu.sync_copy(dstp.at[pl.ds(base, K2), :], dst_v)

    bufs = (rows0, rows1, rows2, rows3)
    sems = (sem0, sem1, sem2, sem3)
    for b in range(3):
      pltpu.async_copy(tab_sh.at[src_v.at[b]], bufs[b], sems[b])

    def quad(t, carry):
      for b in range(4):
        jj = 4 * t + b
        pltpu.make_async_copy(tab_sh.at[src_v.at[jj]], bufs[b], sems[b]).wait()

        @pl.when(jj + 3 < K2)
        def _():
          bn = (b + 3) % 4
          pltpu.async_copy(tab_sh.at[src_v.at[jj + 3]], bufs[bn], sems[bn])

        pltpu.sync_copy(bufs[b], agg_sh.at[dst_v.at[jj]], add=True)
        do_cnt = lax.select(c == 0, jj < K, jj >= K)

        @pl.when(do_cnt)
        def _():
          pltpu.sync_copy(ones_v, cnt_sh.at[dst_v.at[jj]], add=True)
      return carry

    lax.fori_loop(0, K2 // 4, quad, 0)
    plsc.subcore_barrier()
    pltpu.sync_copy(agg_sh.at[pl.ds(r0, RB), :],
                    out_agg.at[c].at[pl.ds(r0, RB), :])
    pltpu.sync_copy(cnt_sh.at[pl.ds(r0, RB), :],
                    out_cnt.at[c].at[pl.ds(r0, RB), :])

  return pl.kernel(body, out_type=out_type, mesh=mesh, scratch_types=scratch,
                   compiler_params=pltpu.CompilerParams(
                       use_tc_tiling_on_sc=False))


def _edge_kernel_l2(Dm):
  mesh = plsc.VectorSubcoreMesh(
      core_axis_name="c", subcore_axis_name="s", num_cores=NC, num_subcores=NS)
  out_type = jax.ShapeDtypeStruct((NC, NPAD, Dm), jnp.float32)
  scratch = [
      pltpu.VMEM((K, CH), jnp.int32),
      pltpu.VMEM((K, CH), jnp.int32),
      pltpu.VMEM((CH, Dm), jnp.float32),
      pltpu.VMEM((CH, Dm), jnp.float32),
      pltpu.VMEM((CH, Dm), jnp.float32),
      pltpu.VMEM((CH, Dm), jnp.float32),
      pltpu.VMEM_SHARED((NPAD, Dm), jnp.float32),
      pltpu.VMEM_SHARED((NPAD, Dm), jnp.float32),
      pltpu.SemaphoreType.DMA,
      pltpu.SemaphoreType.DMA,
      pltpu.SemaphoreType.DMA,
      pltpu.SemaphoreType.DMA,
  ]

  def body(tab, srcp, dstp, z2, out_agg,
           src_v, dst_v, rows0, rows1, rows2, rows3, agg_sh, tab_sh,
           sem0, sem1, sem2, sem3):
    c = lax.axis_index("c")
    s = lax.axis_index("s")
    wid = c * NS + s
    r0 = s * RB
    pltpu.sync_copy(z2.at[pl.ds(r0, RB), :], agg_sh.at[pl.ds(r0, RB), :])
    pltpu.sync_copy(tab.at[pl.ds(r0, RB), :], tab_sh.at[pl.ds(r0, RB), :])
    plsc.subcore_barrier()

    base = wid * K
    pltpu.sync_copy(srcp.at[pl.ds(base, K), :], src_v)
    pltpu.sync_copy(dstp.at[pl.ds(base, K), :], dst_v)

    bufs = (rows0, rows1, rows2, rows3)
    sems = (sem0, sem1, sem2, sem3)
    for b in range(3):
      pltpu.async_copy(tab_sh.at[src_v.at[b]], bufs[b], sems[b])

    def quad(t, carry):
      for b in range(4):
        jj = 4 * t + b
        pltpu.make_async_copy(tab_sh.at[src_v.at[jj]], bufs[b], sems[b]).wait()

        @pl.when(jj + 3 < K)
        def _():
          bn = (b + 3) % 4
          pltpu.async_copy(tab_sh.at[src_v.at[jj + 3]], bufs[bn], sems[bn])

        pltpu.sync_copy(bufs[b], agg_sh.at[dst_v.at[jj]], add=True)
      return carry

    lax.fori_loop(0, K // 4, quad, 0)
    plsc.subcore_barrier()
    pltpu.sync_copy(agg_sh.at[pl.ds(r0, RB), :],
                    out_agg.at[c].at[pl.ds(r0, RB), :])

  return pl.kernel(body, out_type=out_type, mesh=mesh, scratch_types=scratch,
                   compiler_params=pltpu.CompilerParams(
                       use_tc_tiling_on_sc=False))


def _tc1_body(x, W1a, W1b, b1a, b1b, out_a, out_b):
  out_a[...] = jnp.maximum(
      jnp.dot(x[...], W1a[...], preferred_element_type=jnp.float32) + b1a[...],
      0.0)
  out_b[...] = jnp.maximum(
      jnp.dot(x[...], W1b[...], preferred_element_type=jnp.float32) + b1b[...],
      0.0)


def _tc2_body(aggp, cntp, x, W2a, W2b, Wr, b2, W1n, b1n,
              h1, p2, inv, s_a, s_b, s_c0, s_c1):
  i = pl.program_id(0)
  pltpu.sync_copy(aggp.at[0].at[pl.ds(i * RBLK, RBLK), :], s_a)
  pltpu.sync_copy(aggp.at[1].at[pl.ds(i * RBLK, RBLK), :], s_b)
  pltpu.sync_copy(cntp.at[0].at[pl.ds(i * RBLK, RBLK), :], s_c0)
  pltpu.sync_copy(cntp.at[1].at[pl.ds(i * RBLK, RBLK), :], s_c1)
  iv = 1.0 / jnp.maximum(s_c0[:, 0:1] + s_c1[:, 0:1], 1.0)
  h = jnp.maximum(
      jnp.dot(s_a[...] * iv, W2a[...], preferred_element_type=jnp.float32)
      + jnp.dot(s_b[...] * iv, W2b[...], preferred_element_type=jnp.float32)
      + jnp.dot(x[...], Wr[...], preferred_element_type=jnp.float32)
      + b2[...], 0.0)
  h1[...] = h
  p2[...] = jnp.maximum(
      jnp.dot(h, W1n[...], preferred_element_type=jnp.float32) + b1n[...], 0.0)
  inv[...] = iv


def _tc3_body(aggp, inv, h1, W2, Wr, b2, lW, lb, y, s_a, s_b):
  i = pl.program_id(0)
  pltpu.sync_copy(aggp.at[0].at[pl.ds(i * RBLK, RBLK), :], s_a)
  pltpu.sync_copy(aggp.at[1].at[pl.ds(i * RBLK, RBLK), :], s_b)
  mean = (s_a[...] + s_b[...]) * inv[...]
  h = jnp.maximum(
      jnp.dot(mean, W2[...], preferred_element_type=jnp.float32)
      + jnp.dot(h1[...], Wr[...], preferred_element_type=jnp.float32)
      + b2[...], 0.0)
  y[...] = jnp.dot(h, lW[...], preferred_element_type=jnp.float32) + lb[...]


def _row_spec(d):
  return pl.BlockSpec((RBLK, d), lambda i: (i, 0))


def _full_spec(a, b):
  return pl.BlockSpec((a, b), lambda i: (0, 0))


@jax.jit
def kernel(x, edge_index, c1_W1, c1_b1, c1_W2, c1_b2, c1_Wr,
           c2_W1, c2_b1, c2_W2, c2_b2, c2_Wr, lin_W, lin_b):
  E = edge_index.shape[1]
  pad = EP - E
  src = jnp.concatenate([edge_index[0], jnp.zeros((pad,), jnp.int32)])
  dst = jnp.concatenate([edge_index[1], jnp.full((pad,), N, jnp.int32)])
  srcp = src.reshape(NW * K, CH)
  dstp = dst.reshape(NW * K, CH)
  z32 = jnp.zeros((NPAD, 32), jnp.float32)
  z1 = jnp.zeros((NPAD, 8), jnp.float32)
  ones1 = jnp.ones((CH, 8), jnp.float32)

  grid = N // RBLK

  tab_a, tab_b = pl.pallas_call(
      _tc1_body,
      grid=(8,),
      in_specs=[pl.BlockSpec((NPAD // 8, 128), lambda i: (i, 0)),
                _full_spec(128, 32), _full_spec(128, 32),
                _full_spec(1, 32), _full_spec(1, 32)],
      out_specs=[pl.BlockSpec((NPAD // 8, 32), lambda i: (i, 0))] * 2,
      out_shape=[jax.ShapeDtypeStruct((NPAD, 32), jnp.float32)] * 2,
  )(x, c1_W1[:, :32], c1_W1[:, 32:], c1_b1[:32].reshape(1, 32),
    c1_b1[32:].reshape(1, 32))

  agg1p, cntp = _edge_kernel_l1()(tab_a, tab_b, srcp, dstp, z32, z1, ones1)

  any_spec = pl.BlockSpec(memory_space=pl.ANY)

  h1, p2, inv = pl.pallas_call(
      _tc2_body,
      grid=(grid,),
      in_specs=[any_spec, any_spec,
                _row_spec(128), _full_spec(32, 64), _full_spec(32, 64),
                _full_spec(128, 64), _full_spec(1, 64),
                _full_spec(64, 32), _full_spec(1, 32)],
      out_specs=[_row_spec(64), _row_spec(32), _row_spec(1)],
      out_shape=[jax.ShapeDtypeStruct((N, 64), jnp.float32),
                 jax.ShapeDtypeStruct((NPAD, 32), jnp.float32),
                 jax.ShapeDtypeStruct((N, 1), jnp.float32)],
      scratch_shapes=[pltpu.VMEM((RBLK, 32), jnp.float32),
                      pltpu.VMEM((RBLK, 32), jnp.float32),
                      pltpu.VMEM((RBLK, 8), jnp.float32),
                      pltpu.VMEM((RBLK, 8), jnp.float32)],
  )(agg1p, cntp, x, c1_W2[:32, :], c1_W2[32:, :], c1_Wr,
    c1_b2.reshape(1, 64), c2_W1, c2_b1.reshape(1, 32))

  agg2p = _edge_kernel_l2(32)(p2, srcp, dstp, z32)

  y = pl.pallas_call(
      _tc3_body,
      grid=(grid,),
      in_specs=[any_spec, _row_spec(1), _row_spec(64),
                _full_spec(32, 32), _full_spec(64, 32), _full_spec(1, 32),
                _full_spec(32, 1), _full_spec(1, 1)],
      out_specs=_row_spec(1),
      out_shape=jax.ShapeDtypeStruct((N, 1), jnp.float32),
      scratch_shapes=[pltpu.VMEM((RBLK, 32), jnp.float32),
                      pltpu.VMEM((RBLK, 32), jnp.float32)],
  )(agg2p, inv, h1, c2_W2, c2_Wr, c2_b2.reshape(1, 32),
    lin_W, lin_b.reshape(1, 1))

  return y


# speedup vs baseline: 1.1370x; 1.1370x over previous
import functools

import jax
import jax.numpy as jnp
from jax import lax
from jax.experimental import pallas as pl
from jax.experimental.pallas import tpu as pltpu
from jax.experimental.pallas import tpu_sc as plsc

N = 10000
NC = 2
NS = 16
NW = NC * NS
CH = 128
K = 80
K2 = 2 * K
EP = NW * K * CH
RB = 632
NPAD = NS * RB
RBLK = 2000


def _edge_kernel_l1():
  mesh = plsc.VectorSubcoreMesh(
      core_axis_name="c", subcore_axis_name="s", num_cores=NC, num_subcores=NS)
  out_type = [jax.ShapeDtypeStruct((NC, NPAD, 32), jnp.float32),
              jax.ShapeDtypeStruct((NC, NPAD, 8), jnp.float32)]
  scratch = [
      pltpu.VMEM((K2, CH), jnp.int32),
      pltpu.VMEM((K2, CH), jnp.int32),
      pltpu.VMEM((CH, 32), jnp.float32),
      pltpu.VMEM((CH, 32), jnp.float32),
      pltpu.VMEM((CH, 32), jnp.float32),
      pltpu.VMEM((CH, 32), jnp.float32),
      pltpu.VMEM((CH, 8), jnp.float32),
      pltpu.VMEM_SHARED((NPAD, 32), jnp.float32),
      pltpu.VMEM_SHARED((NPAD, 32), jnp.float32),
      pltpu.VMEM_SHARED((NPAD, 8), jnp.float32),
      pltpu.SemaphoreType.DMA,
      pltpu.SemaphoreType.DMA,
      pltpu.SemaphoreType.DMA,
      pltpu.SemaphoreType.DMA,
  ]

  def body(tab_a, tab_b, srcp, dstp, z32, z1, ones_h,
           out_agg, out_cnt,
           src_v, dst_v, rows0, rows1, rows2, rows3, ones_v,
           agg_sh, tab_sh, cnt_sh, sem0, sem1, sem2, sem3):
    c = lax.axis_index("c")
    s = lax.axis_index("s")
    r0 = s * RB
    pltpu.sync_copy(z32.at[pl.ds(r0, RB), :], agg_sh.at[pl.ds(r0, RB), :])
    pltpu.sync_copy(z1.at[pl.ds(r0, RB), :], cnt_sh.at[pl.ds(r0, RB), :])
    pltpu.sync_copy(ones_h, ones_v)

    @pl.when(c == 0)
    def _():
      pltpu.sync_copy(tab_a.at[pl.ds(r0, RB), :], tab_sh.at[pl.ds(r0, RB), :])

    @pl.when(c == 1)
    def _():
      pltpu.sync_copy(tab_b.at[pl.ds(r0, RB), :], tab_sh.at[pl.ds(r0, RB), :])

    plsc.subcore_barrier()

    base = s * K2
    pltpu.sync_copy(srcp.at[pl.ds(base, K2), :], src_v)
    pltpu.sync_copy(dstp.at[pl.ds(base, K2), :], dst_v)

    bufs = (rows0, rows1, rows2, rows3)
    sems = (sem0, sem1, sem2, sem3)
    for b in range(3):
      pltpu.async_copy(tab_sh.at[src_v.at[b]], bufs[b], sems[b])

    def quad(t, carry):
      for b in range(4):
        jj = 4 * t + b
        pltpu.make_async_copy(tab_sh.at[src_v.at[jj]], bufs[b], sems[b]).wait()

        @pl.when(jj + 3 < K2)
        def _():
          bn = (b + 3) % 4
          pltpu.async_copy(tab_sh.at[src_v.at[jj + 3]], bufs[bn], sems[bn])

        pltpu.sync_copy(bufs[b], agg_sh.at[dst_v.at[jj]], add=True)
        do_cnt = lax.select(c == 0, jj < K, jj >= K)

        @pl.when(do_cnt)
        def _():
          pltpu.sync_copy(ones_v, cnt_sh.at[dst_v.at[jj]], add=True)
      return carry

    lax.fori_loop(0, K2 // 4, quad, 0)
    plsc.subcore_barrier()
    pltpu.sync_copy(agg_sh.at[pl.ds(r0, RB), :],
                    out_agg.at[c].at[pl.ds(r0, RB), :])
    pltpu.sync_copy(cnt_sh.at[pl.ds(r0, RB), :],
                    out_cnt.at[c].at[pl.ds(r0, RB), :])

  return pl.kernel(body, out_type=out_type, mesh=mesh, scratch_types=scratch,
                   compiler_params=pltpu.CompilerParams(
                       use_tc_tiling_on_sc=False))


def _edge_kernel_l2(Dm):
  mesh = plsc.VectorSubcoreMesh(
      core_axis_name="c", subcore_axis_name="s", num_cores=NC, num_subcores=NS)
  out_type = jax.ShapeDtypeStruct((NC, NPAD, Dm), jnp.float32)
  scratch = [
      pltpu.VMEM((K, CH), jnp.int32),
      pltpu.VMEM((K, CH), jnp.int32),
      pltpu.VMEM((CH, Dm), jnp.float32),
      pltpu.VMEM((CH, Dm), jnp.float32),
      pltpu.VMEM((CH, Dm), jnp.float32),
      pltpu.VMEM((CH, Dm), jnp.float32),
      pltpu.VMEM_SHARED((NPAD, Dm), jnp.float32),
      pltpu.VMEM_SHARED((NPAD, Dm), jnp.float32),
      pltpu.SemaphoreType.DMA,
      pltpu.SemaphoreType.DMA,
      pltpu.SemaphoreType.DMA,
      pltpu.SemaphoreType.DMA,
  ]

  def body(tab, srcp, dstp, z2, out_agg,
           src_v, dst_v, rows0, rows1, rows2, rows3, agg_sh, tab_sh,
           sem0, sem1, sem2, sem3):
    c = lax.axis_index("c")
    s = lax.axis_index("s")
    wid = c * NS + s
    r0 = s * RB
    pltpu.sync_copy(z2.at[pl.ds(r0, RB), :], agg_sh.at[pl.ds(r0, RB), :])
    pltpu.sync_copy(tab.at[pl.ds(r0, RB), :], tab_sh.at[pl.ds(r0, RB), :])
    plsc.subcore_barrier()

    base = wid * K
    pltpu.sync_copy(srcp.at[pl.ds(base, K), :], src_v)
    pltpu.sync_copy(dstp.at[pl.ds(base, K), :], dst_v)

    bufs = (rows0, rows1, rows2, rows3)
    sems = (sem0, sem1, sem2, sem3)
    for b in range(3):
      pltpu.async_copy(tab_sh.at[src_v.at[b]], bufs[b], sems[b])

    def quad(t, carry):
      for b in range(4):
        jj = 4 * t + b
        pltpu.make_async_copy(tab_sh.at[src_v.at[jj]], bufs[b], sems[b]).wait()

        @pl.when(jj + 3 < K)
        def _():
          bn = (b + 3) % 4
          pltpu.async_copy(tab_sh.at[src_v.at[jj + 3]], bufs[bn], sems[bn])

        pltpu.sync_copy(bufs[b], agg_sh.at[dst_v.at[jj]], add=True)
      return carry

    lax.fori_loop(0, K // 4, quad, 0)
    plsc.subcore_barrier()
    pltpu.sync_copy(agg_sh.at[pl.ds(r0, RB), :],
                    out_agg.at[c].at[pl.ds(r0, RB), :])

  return pl.kernel(body, out_type=out_type, mesh=mesh, scratch_types=scratch,
                   compiler_params=pltpu.CompilerParams(
                       use_tc_tiling_on_sc=False))


def _tc1_body(x, W1a, W1b, b1a, b1b, out_a, out_b):
  out_a[...] = jnp.maximum(
      jnp.dot(x[...], W1a[...], preferred_element_type=jnp.float32) + b1a[...],
      0.0)
  out_b[...] = jnp.maximum(
      jnp.dot(x[...], W1b[...], preferred_element_type=jnp.float32) + b1b[...],
      0.0)


def _tc2_body(agg_a, agg_b, cnt0, cnt1, x, W2a, W2b, Wr, b2, W1n, b1n,
              h1, p2, inv):
  iv = 1.0 / jnp.maximum(cnt0[0, :, 0:1] + cnt1[0, :, 0:1], 1.0)
  h = jnp.maximum(
      jnp.dot(agg_a[0] * iv, W2a[...], preferred_element_type=jnp.float32)
      + jnp.dot(agg_b[0] * iv, W2b[...], preferred_element_type=jnp.float32)
      + jnp.dot(x[...], Wr[...], preferred_element_type=jnp.float32)
      + b2[...], 0.0)
  h1[...] = h
  p2[...] = jnp.maximum(
      jnp.dot(h, W1n[...], preferred_element_type=jnp.float32) + b1n[...], 0.0)
  inv[...] = iv


def _tc3_body(agg0, agg1, inv, h1, W2, Wr, b2, lW, lb, y):
  mean = (agg0[0] + agg1[0]) * inv[...]
  h = jnp.maximum(
      jnp.dot(mean, W2[...], preferred_element_type=jnp.float32)
      + jnp.dot(h1[...], Wr[...], preferred_element_type=jnp.float32)
      + b2[...], 0.0)
  y[...] = jnp.dot(h, lW[...], preferred_element_type=jnp.float32) + lb[...]


def _row_spec(d):
  return pl.BlockSpec((RBLK, d), lambda i: (i, 0))


def _full_spec(a, b):
  return pl.BlockSpec((a, b), lambda i: (0, 0))


@jax.jit
def kernel(x, edge_index, c1_W1, c1_b1, c1_W2, c1_b2, c1_Wr,
           c2_W1, c2_b1, c2_W2, c2_b2, c2_Wr, lin_W, lin_b):
  E = edge_index.shape[1]
  pad = EP - E
  src = jnp.concatenate([edge_index[0], jnp.zeros((pad,), jnp.int32)])
  dst = jnp.concatenate([edge_index[1], jnp.full((pad,), N, jnp.int32)])
  srcp = src.reshape(NW * K, CH)
  dstp = dst.reshape(NW * K, CH)
  z32 = jnp.zeros((NPAD, 32), jnp.float32)
  z1 = jnp.zeros((NPAD, 8), jnp.float32)
  ones1 = jnp.ones((CH, 8), jnp.float32)

  grid = N // RBLK

  tab_a, tab_b = pl.pallas_call(
      _tc1_body,
      grid=(8,),
      in_specs=[pl.BlockSpec((NPAD // 8, 128), lambda i: (i, 0)),
                _full_spec(128, 32), _full_spec(128, 32),
                _full_spec(1, 32), _full_spec(1, 32)],
      out_specs=[pl.BlockSpec((NPAD // 8, 32), lambda i: (i, 0))] * 2,
      out_shape=[jax.ShapeDtypeStruct((NPAD, 32), jnp.float32)] * 2,
  )(x, c1_W1[:, :32], c1_W1[:, 32:], c1_b1[:32].reshape(1, 32),
    c1_b1[32:].reshape(1, 32))

  agg1p, cntp = _edge_kernel_l1()(tab_a, tab_b, srcp, dstp, z32, z1, ones1)

  def _sc_spec(d, c):
    return pl.BlockSpec((1, RBLK, d), lambda i, c=c: (c, i, 0))

  h1, p2, inv = pl.pallas_call(
      _tc2_body,
      grid=(grid,),
      in_specs=[_sc_spec(32, 0), _sc_spec(32, 1), _sc_spec(8, 0),
                _sc_spec(8, 1),
                _row_spec(128), _full_spec(32, 64), _full_spec(32, 64),
                _full_spec(128, 64), _full_spec(1, 64),
                _full_spec(64, 32), _full_spec(1, 32)],
      out_specs=[_row_spec(64), _row_spec(32), _row_spec(1)],
      out_shape=[jax.ShapeDtypeStruct((N, 64), jnp.float32),
                 jax.ShapeDtypeStruct((NPAD, 32), jnp.float32),
                 jax.ShapeDtypeStruct((N, 1), jnp.float32)],
  )(agg1p, agg1p, cntp, cntp, x, c1_W2[:32, :], c1_W2[32:, :], c1_Wr,
    c1_b2.reshape(1, 64), c2_W1, c2_b1.reshape(1, 32))

  agg2p = _edge_kernel_l2(32)(p2, srcp, dstp, z32)

  y = pl.pallas_call(
      _tc3_body,
      grid=(grid,),
      in_specs=[_sc_spec(32, 0), _sc_spec(32, 1), _row_spec(1), _row_spec(64),
                _full_spec(32, 32), _full_spec(64, 32), _full_spec(1, 32),
                _full_spec(32, 1), _full_spec(1, 1)],
      out_specs=_row_spec(1),
      out_shape=jax.ShapeDtypeStruct((N, 1), jnp.float32),
  )(agg2p, agg2p, inv, h1, c2_W2, c2_Wr, c2_b2.reshape(1, 32),
    lin_W, lin_b.reshape(1, 1))

  return y
